# baseline (speedup 1.0000x reference)
import functools

import jax
import jax.numpy as jnp
from jax import lax
from jax.experimental import pallas as pl
from jax.experimental.pallas import tpu as pltpu
from jax.experimental.pallas import tpu_sc as plsc

N = 10000
NC = 2
NS = 16
NW = NC * NS
CH = 128
K = 80
EP = NW * K * CH
RB = 632
NPAD = NS * RB
RBLK = 2000
D1 = 80


def _edge_kernel(Dm):
  mesh = plsc.VectorSubcoreMesh(
      core_axis_name="c", subcore_axis_name="s", num_cores=NC, num_subcores=NS)
  out_type = jax.ShapeDtypeStruct((NC, NPAD, Dm), jnp.float32)
  scratch = [
      pltpu.VMEM((K, CH), jnp.int32),
      pltpu.VMEM((K, CH), jnp.int32),
      pltpu.VMEM((CH, Dm), jnp.float32),
      pltpu.VMEM((CH, Dm), jnp.float32),
      pltpu.VMEM_SHARED((NPAD, Dm), jnp.float32),
      pltpu.SemaphoreType.DMA,
      pltpu.SemaphoreType.DMA,
  ]

  def body(tab, srcp, dstp, z2, out_agg,
           src_v, dst_v, rows0, rows1, agg_sh, sem0, sem1):
    c = lax.axis_index("c")
    s = lax.axis_index("s")
    wid = c * NS + s
    r0 = s * RB
    pltpu.sync_copy(z2.at[pl.ds(r0, RB), :], agg_sh.at[pl.ds(r0, RB), :])
    plsc.subcore_barrier()

    base = wid * K
    pltpu.sync_copy(srcp.at[pl.ds(base, K), :], src_v)
    pltpu.sync_copy(dstp.at[pl.ds(base, K), :], dst_v)

    pltpu.async_copy(tab.at[src_v.at[0]], rows0, sem0)

    def pair(t, carry):
      jj = 2 * t
      pltpu.make_async_copy(tab.at[src_v.at[jj]], rows0, sem0).wait()
      pltpu.async_copy(tab.at[src_v.at[jj + 1]], rows1, sem1)
      pltpu.sync_copy(rows0, agg_sh.at[dst_v.at[jj]], add=True)
      pltpu.make_async_copy(tab.at[src_v.at[jj + 1]], rows1, sem1).wait()

      @pl.when(jj + 2 < K)
      def _():
        pltpu.async_copy(tab.at[src_v.at[jj + 2]], rows0, sem0)

      pltpu.sync_copy(rows1, agg_sh.at[dst_v.at[jj + 1]], add=True)
      return carry

    lax.fori_loop(0, K // 2, pair, 0)
    plsc.subcore_barrier()
    pltpu.sync_copy(agg_sh.at[pl.ds(r0, RB), :],
                    out_agg.at[c].at[pl.ds(r0, RB), :])

  return pl.kernel(body, out_type=out_type, mesh=mesh, scratch_types=scratch,
                   compiler_params=pltpu.CompilerParams(
                       use_tc_tiling_on_sc=False))


def _tc1_body(x, W1, b1, out):
  out[...] = jnp.maximum(
      jnp.dot(x[...], W1[...], preferred_element_type=jnp.float32) + b1[...],
      0.0)


def _tc2_body(agg0, agg1, cnt0, cnt1, x, W2, Wr, b2, W1n, b1n,
              h1, p2, inv):
  iv = 1.0 / jnp.maximum(cnt0[...] + cnt1[...], 1.0)
  mean = (agg0[...] + agg1[...]) * iv
  h = jnp.maximum(
      jnp.dot(mean, W2[...], preferred_element_type=jnp.float32)
      + jnp.dot(x[...], Wr[...], preferred_element_type=jnp.float32)
      + b2[...], 0.0)
  h1[...] = h
  p2[...] = jnp.maximum(
      jnp.dot(h, W1n[...], preferred_element_type=jnp.float32) + b1n[...], 0.0)
  inv[...] = iv


def _tc3_body(agg0, agg1, inv, h1, W2, Wr, b2, lW, lb, y):
  mean = (agg0[...] + agg1[...]) * inv[...]
  h = jnp.maximum(
      jnp.dot(mean, W2[...], preferred_element_type=jnp.float32)
      + jnp.dot(h1[...], Wr[...], preferred_element_type=jnp.float32)
      + b2[...], 0.0)
  y[...] = jnp.dot(h, lW[...], preferred_element_type=jnp.float32) + lb[...]


def _row_spec(d):
  return pl.BlockSpec((RBLK, d), lambda i: (i, 0))


def _full_spec(a, b):
  return pl.BlockSpec((a, b), lambda i: (0, 0))


@jax.jit
def kernel(x, edge_index, c1_W1, c1_b1, c1_W2, c1_b2, c1_Wr,
           c2_W1, c2_b1, c2_W2, c2_b2, c2_Wr, lin_W, lin_b):
  E = edge_index.shape[1]
  pad = EP - E
  src = jnp.concatenate([edge_index[0], jnp.zeros((pad,), jnp.int32)])
  dst = jnp.concatenate([edge_index[1], jnp.full((pad,), N, jnp.int32)])
  srcp = src.reshape(NW * K, CH)
  dstp = dst.reshape(NW * K, CH)
  zD1 = jnp.zeros((NPAD, D1), jnp.float32)
  z32 = jnp.zeros((NPAD, 32), jnp.float32)

  W1aug = jnp.concatenate(
      [c1_W1, jnp.zeros((128, D1 - 64), jnp.float32)], axis=1)
  b1aug = jnp.concatenate(
      [c1_b1, jnp.ones((1,), jnp.float32),
       jnp.zeros((D1 - 65,), jnp.float32)])
  W2aug = jnp.concatenate(
      [c1_W2, jnp.zeros((D1 - 64, 64), jnp.float32)], axis=0)

  grid = N // RBLK

  p1 = pl.pallas_call(
      _tc1_body,
      grid=(grid,),
      in_specs=[_row_spec(128), _full_spec(128, D1), _full_spec(1, D1)],
      out_specs=_row_spec(D1),
      out_shape=jax.ShapeDtypeStruct((N, D1), jnp.float32),
  )(x, W1aug, b1aug.reshape(1, D1))

  agg1p = _edge_kernel(D1)(p1, srcp, dstp, zD1)
  agg1_0 = agg1p[0, :N, :]
  agg1_1 = agg1p[1, :N, :]
  cnt0 = agg1p[0, :N, 64:65]
  cnt1 = agg1p[1, :N, 64:65]

  h1, p2, inv = pl.pallas_call(
      _tc2_body,
      grid=(grid,),
      in_specs=[_row_spec(D1), _row_spec(D1), _row_spec(1), _row_spec(1),
                _row_spec(128), _full_spec(D1, 64), _full_spec(128, 64),
                _full_spec(1, 64), _full_spec(64, 32), _full_spec(1, 32)],
      out_specs=[_row_spec(64), _row_spec(32), _row_spec(1)],
      out_shape=[jax.ShapeDtypeStruct((N, 64), jnp.float32),
                 jax.ShapeDtypeStruct((N, 32), jnp.float32),
                 jax.ShapeDtypeStruct((N, 1), jnp.float32)],
  )(agg1_0, agg1_1, cnt0, cnt1, x, W2aug, c1_Wr, c1_b2.reshape(1, 64),
    c2_W1, c2_b1.reshape(1, 32))

  agg2p = _edge_kernel(32)(p2, srcp, dstp, z32)
  agg2_0 = agg2p[0, :N, :]
  agg2_1 = agg2p[1, :N, :]

  y = pl.pallas_call(
      _tc3_body,
      grid=(grid,),
      in_specs=[_row_spec(32), _row_spec(32), _row_spec(1), _row_spec(64),
                _full_spec(32, 32), _full_spec(64, 32), _full_spec(1, 32),
                _full_spec(32, 1), _full_spec(1, 1)],
      out_specs=_row_spec(1),
      out_shape=jax.ShapeDtypeStruct((N, 1), jnp.float32),
  )(agg2_0, agg2_1, inv, h1, c2_W2, c2_Wr, c2_b2.reshape(1, 32),
    lin_W, lin_b.reshape(1, 1))

  return y


# speedup vs baseline: 1.3470x; 1.3470x over previous
import functools

import jax
import jax.numpy as jnp
from jax import lax
from jax.experimental import pallas as pl
from jax.experimental.pallas import tpu as pltpu
from jax.experimental.pallas import tpu_sc as plsc

N = 10000
NC = 2
NS = 16
NW = NC * NS
CH = 128
K = 80
EP = NW * K * CH
RB = 632
NPAD = NS * RB
RBLK = 2000


def _edge_kernel(Dm, with_count):
  mesh = plsc.VectorSubcoreMesh(
      core_axis_name="c", subcore_axis_name="s", num_cores=NC, num_subcores=NS)
  out_type = [jax.ShapeDtypeStruct((NC, NPAD, Dm), jnp.float32)]
  scratch = [
      pltpu.VMEM((K, CH), jnp.int32),
      pltpu.VMEM((K, CH), jnp.int32),
      pltpu.VMEM((CH, Dm), jnp.float32),
      pltpu.VMEM((CH, Dm), jnp.float32),
      pltpu.VMEM((CH, Dm), jnp.float32),
      pltpu.VMEM((CH, Dm), jnp.float32),
      pltpu.VMEM_SHARED((NPAD, Dm), jnp.float32),
      pltpu.SemaphoreType.DMA,
      pltpu.SemaphoreType.DMA,
      pltpu.SemaphoreType.DMA,
      pltpu.SemaphoreType.DMA,
  ]
  if with_count:
    out_type.append(jax.ShapeDtypeStruct((NC, NPAD, 8), jnp.float32))
    scratch += [
        pltpu.VMEM((CH, 8), jnp.float32),
        pltpu.VMEM_SHARED((NPAD, 8), jnp.float32),
    ]

  def body(*refs):
    if with_count:
      (tab, srcp, dstp, z2, z8, ones_h,
       out_agg, out_cnt,
       src_v, dst_v, rows0, rows1, rows2, rows3, agg_sh,
       sem0, sem1, sem2, sem3, ones_v, cnt_sh) = refs
    else:
      (tab, srcp, dstp, z2,
       out_agg,
       src_v, dst_v, rows0, rows1, rows2, rows3, agg_sh,
       sem0, sem1, sem2, sem3) = refs
    c = lax.axis_index("c")
    s = lax.axis_index("s")
    wid = c * NS + s
    r0 = s * RB
    pltpu.sync_copy(z2.at[pl.ds(r0, RB), :], agg_sh.at[pl.ds(r0, RB), :])
    if with_count:
      pltpu.sync_copy(z8.at[pl.ds(r0, RB), :], cnt_sh.at[pl.ds(r0, RB), :])
      pltpu.sync_copy(ones_h, ones_v)
    plsc.subcore_barrier()

    base = wid * K
    pltpu.sync_copy(srcp.at[pl.ds(base, K), :], src_v)
    pltpu.sync_copy(dstp.at[pl.ds(base, K), :], dst_v)

    bufs = (rows0, rows1, rows2, rows3)
    sems = (sem0, sem1, sem2, sem3)
    for b in range(3):
      pltpu.async_copy(tab.at[src_v.at[b]], bufs[b], sems[b])

    def quad(t, carry):
      for b in range(4):
        jj = 4 * t + b
        pltpu.make_async_copy(tab.at[src_v.at[jj]], bufs[b], sems[b]).wait()

        @pl.when(jj + 3 < K)
        def _():
          bn = (b + 3) % 4
          pltpu.async_copy(tab.at[src_v.at[jj + 3]], bufs[bn], sems[bn])

        pltpu.sync_copy(bufs[b], agg_sh.at[dst_v.at[jj]], add=True)
        if with_count:
          pltpu.sync_copy(ones_v, cnt_sh.at[dst_v.at[jj]], add=True)
      return carry

    lax.fori_loop(0, K // 4, quad, 0)
    plsc.subcore_barrier()
    pltpu.sync_copy(agg_sh.at[pl.ds(r0, RB), :],
                    out_agg.at[c].at[pl.ds(r0, RB), :])
    if with_count:
      pltpu.sync_copy(cnt_sh.at[pl.ds(r0, RB), :],
                      out_cnt.at[c].at[pl.ds(r0, RB), :])

  return pl.kernel(body, out_type=out_type, mesh=mesh, scratch_types=scratch,
                   compiler_params=pltpu.CompilerParams(
                       use_tc_tiling_on_sc=False))


def _tc1_body(x, W1, b1, out):
  out[...] = jnp.maximum(
      jnp.dot(x[...], W1[...], preferred_element_type=jnp.float32) + b1[...],
      0.0)


def _tc2_body(agg0, agg1, cnt0, cnt1, x, W2, Wr, b2, W1n, b1n,
              h1, p2, inv):
  iv = 1.0 / jnp.maximum(cnt0[...] + cnt1[...], 1.0)
  mean = (agg0[...] + agg1[...]) * iv
  h = jnp.maximum(
      jnp.dot(mean, W2[...], preferred_element_type=jnp.float32)
      + jnp.dot(x[...], Wr[...], preferred_element_type=jnp.float32)
      + b2[...], 0.0)
  h1[...] = h
  p2[...] = jnp.maximum(
      jnp.dot(h, W1n[...], preferred_element_type=jnp.float32) + b1n[...], 0.0)
  inv[...] = iv


def _tc3_body(agg0, agg1, inv, h1, W2, Wr, b2, lW, lb, y):
  mean = (agg0[...] + agg1[...]) * inv[...]
  h = jnp.maximum(
      jnp.dot(mean, W2[...], preferred_element_type=jnp.float32)
      + jnp.dot(h1[...], Wr[...], preferred_element_type=jnp.float32)
      + b2[...], 0.0)
  y[...] = jnp.dot(h, lW[...], preferred_element_type=jnp.float32) + lb[...]


def _row_spec(d):
  return pl.BlockSpec((RBLK, d), lambda i: (i, 0))


def _full_spec(a, b):
  return pl.BlockSpec((a, b), lambda i: (0, 0))


@jax.jit
def kernel(x, edge_index, c1_W1, c1_b1, c1_W2, c1_b2, c1_Wr,
           c2_W1, c2_b1, c2_W2, c2_b2, c2_Wr, lin_W, lin_b):
  E = edge_index.shape[1]
  pad = EP - E
  src = jnp.concatenate([edge_index[0], jnp.zeros((pad,), jnp.int32)])
  dst = jnp.concatenate([edge_index[1], jnp.full((pad,), N, jnp.int32)])
  srcp = src.reshape(NW * K, CH)
  dstp = dst.reshape(NW * K, CH)
  z64 = jnp.zeros((NPAD, 64), jnp.float32)
  z32 = jnp.zeros((NPAD, 32), jnp.float32)
  z8 = jnp.zeros((NPAD, 8), jnp.float32)
  ones8 = jnp.ones((CH, 8), jnp.float32)

  grid = N // RBLK

  p1 = pl.pallas_call(
      _tc1_body,
      grid=(grid,),
      in_specs=[_row_spec(128), _full_spec(128, 64), _full_spec(1, 64)],
      out_specs=_row_spec(64),
      out_shape=jax.ShapeDtypeStruct((N, 64), jnp.float32),
  )(x, c1_W1, c1_b1.reshape(1, 64))

  agg1p, cntp = _edge_kernel(64, True)(p1, srcp, dstp, z64, z8, ones8)
  agg1_0 = agg1p[0, :N, :]
  agg1_1 = agg1p[1, :N, :]
  cnt0 = cntp[0, :N, 0:1]
  cnt1 = cntp[1, :N, 0:1]

  h1, p2, inv = pl.pallas_call(
      _tc2_body,
      grid=(grid,),
      in_specs=[_row_spec(64), _row_spec(64), _row_spec(1), _row_spec(1),
                _row_spec(128), _full_spec(64, 64), _full_spec(128, 64),
                _full_spec(1, 64), _full_spec(64, 32), _full_spec(1, 32)],
      out_specs=[_row_spec(64), _row_spec(32), _row_spec(1)],
      out_shape=[jax.ShapeDtypeStruct((N, 64), jnp.float32),
                 jax.ShapeDtypeStruct((N, 32), jnp.float32),
                 jax.ShapeDtypeStruct((N, 1), jnp.float32)],
  )(agg1_0, agg1_1, cnt0, cnt1, x, c1_W2, c1_Wr, c1_b2.reshape(1, 64),
    c2_W1, c2_b1.reshape(1, 32))

  (agg2p,) = _edge_kernel(32, False)(p2, srcp, dstp, z32)
  agg2_0 = agg2p[0, :N, :]
  agg2_1 = agg2p[1, :N, :]

  y = pl.pallas_call(
      _tc3_body,
      grid=(grid,),
      in_specs=[_row_spec(32), _row_spec(32), _row_spec(1), _row_spec(64),
                _full_spec(32, 32), _full_spec(64, 32), _full_spec(1, 32),
                _full_spec(32, 1), _full_spec(1, 1)],
      out_specs=_row_spec(1),
      out_shape=jax.ShapeDtypeStruct((N, 1), jnp.float32),
  )(agg2_0, agg2_1, inv, h1, c2_W2, c2_Wr, c2_b2.reshape(1, 32),
    lin_W, lin_b.reshape(1, 1))

  return y


# speedup vs baseline: 1.5633x; 1.1606x over previous
import functools

import jax
import jax.numpy as jnp
from jax import lax
from jax.experimental import pallas as pl
from jax.experimental.pallas import tpu as pltpu
from jax.experimental.pallas import tpu_sc as plsc

N = 10000
NC = 2
NS = 16
NW = NC * NS
CH = 128
K = 80
EP = NW * K * CH
RB = 632
NPAD = NS * RB
RBLK = 2000


def _edge_kernel(Dm, with_count):
  mesh = plsc.VectorSubcoreMesh(
      core_axis_name="c", subcore_axis_name="s", num_cores=NC, num_subcores=NS)
  stage_tab = not with_count
  out_type = [jax.ShapeDtypeStruct((NC, NPAD, Dm), jnp.float32)]
  scratch = [
      pltpu.VMEM((K, CH), jnp.int32),
      pltpu.VMEM((K, CH), jnp.int32),
      pltpu.VMEM((CH, Dm), jnp.float32),
      pltpu.VMEM((CH, Dm), jnp.float32),
      pltpu.VMEM((CH, Dm), jnp.float32),
      pltpu.VMEM((CH, Dm), jnp.float32),
      pltpu.VMEM_SHARED((NPAD, Dm), jnp.float32),
      pltpu.SemaphoreType.DMA,
      pltpu.SemaphoreType.DMA,
      pltpu.SemaphoreType.DMA,
      pltpu.SemaphoreType.DMA,
  ]
  if stage_tab:
    scratch.append(pltpu.VMEM_SHARED((NPAD, Dm), jnp.float32))
  if with_count:
    out_type.append(jax.ShapeDtypeStruct((NC, NPAD, 8), jnp.float32))
    scratch += [
        pltpu.VMEM((CH, 8), jnp.float32),
        pltpu.VMEM_SHARED((NPAD, 8), jnp.float32),
    ]

  def body(*refs):
    if with_count:
      (tab, srcp, dstp, z2, z8, ones_h,
       out_agg, out_cnt,
       src_v, dst_v, rows0, rows1, rows2, rows3, agg_sh,
       sem0, sem1, sem2, sem3, ones_v, cnt_sh) = refs
      gsrc = tab
    else:
      (tab, srcp, dstp, z2,
       out_agg,
       src_v, dst_v, rows0, rows1, rows2, rows3, agg_sh,
       sem0, sem1, sem2, sem3, tab_sh) = refs
      gsrc = tab_sh
    c = lax.axis_index("c")
    s = lax.axis_index("s")
    wid = c * NS + s
    r0 = s * RB
    pltpu.sync_copy(z2.at[pl.ds(r0, RB), :], agg_sh.at[pl.ds(r0, RB), :])
    if stage_tab:
      pltpu.sync_copy(tab.at[pl.ds(r0, RB), :], tab_sh.at[pl.ds(r0, RB), :])
    if with_count:
      pltpu.sync_copy(z8.at[pl.ds(r0, RB), :], cnt_sh.at[pl.ds(r0, RB), :])
      pltpu.sync_copy(ones_h, ones_v)
    plsc.subcore_barrier()

    base = wid * K
    pltpu.sync_copy(srcp.at[pl.ds(base, K), :], src_v)
    pltpu.sync_copy(dstp.at[pl.ds(base, K), :], dst_v)

    bufs = (rows0, rows1, rows2, rows3)
    sems = (sem0, sem1, sem2, sem3)
    for b in range(3):
      pltpu.async_copy(gsrc.at[src_v.at[b]], bufs[b], sems[b])

    def quad(t, carry):
      for b in range(4):
        jj = 4 * t + b
        pltpu.make_async_copy(gsrc.at[src_v.at[jj]], bufs[b], sems[b]).wait()

        @pl.when(jj + 3 < K)
        def _():
          bn = (b + 3) % 4
          pltpu.async_copy(gsrc.at[src_v.at[jj + 3]], bufs[bn], sems[bn])

        pltpu.sync_copy(bufs[b], agg_sh.at[dst_v.at[jj]], add=True)
        if with_count:
          pltpu.sync_copy(ones_v, cnt_sh.at[dst_v.at[jj]], add=True)
      return carry

    lax.fori_loop(0, K // 4, quad, 0)
    plsc.subcore_barrier()
    pltpu.sync_copy(agg_sh.at[pl.ds(r0, RB), :],
                    out_agg.at[c].at[pl.ds(r0, RB), :])
    if with_count:
      pltpu.sync_copy(cnt_sh.at[pl.ds(r0, RB), :],
                      out_cnt.at[c].at[pl.ds(r0, RB), :])

  return pl.kernel(body, out_type=out_type, mesh=mesh, scratch_types=scratch,
                   compiler_params=pltpu.CompilerParams(
                       use_tc_tiling_on_sc=False))


def _tc1_body(x, W1, b1, out):
  out[...] = jnp.maximum(
      jnp.dot(x[...], W1[...], preferred_element_type=jnp.float32) + b1[...],
      0.0)


def _tc2_body(agg0, agg1, cnt0, cnt1, x, W2, Wr, b2, W1n, b1n,
              h1, p2, inv):
  iv = 1.0 / jnp.maximum(cnt0[...] + cnt1[...], 1.0)
  mean = (agg0[...] + agg1[...]) * iv
  h = jnp.maximum(
      jnp.dot(mean, W2[...], preferred_element_type=jnp.float32)
      + jnp.dot(x[...], Wr[...], preferred_element_type=jnp.float32)
      + b2[...], 0.0)
  h1[...] = h
  p2[...] = jnp.maximum(
      jnp.dot(h, W1n[...], preferred_element_type=jnp.float32) + b1n[...], 0.0)
  inv[...] = iv


def _tc3_body(agg0, agg1, inv, h1, W2, Wr, b2, lW, lb, y):
  mean = (agg0[...] + agg1[...]) * inv[...]
  h = jnp.maximum(
      jnp.dot(mean, W2[...], preferred_element_type=jnp.float32)
      + jnp.dot(h1[...], Wr[...], preferred_element_type=jnp.float32)
      + b2[...], 0.0)
  y[...] = jnp.dot(h, lW[...], preferred_element_type=jnp.float32) + lb[...]


def _row_spec(d):
  return pl.BlockSpec((RBLK, d), lambda i: (i, 0))


def _full_spec(a, b):
  return pl.BlockSpec((a, b), lambda i: (0, 0))


@jax.jit
def kernel(x, edge_index, c1_W1, c1_b1, c1_W2, c1_b2, c1_Wr,
           c2_W1, c2_b1, c2_W2, c2_b2, c2_Wr, lin_W, lin_b):
  E = edge_index.shape[1]
  pad = EP - E
  src = jnp.concatenate([edge_index[0], jnp.zeros((pad,), jnp.int32)])
  dst = jnp.concatenate([edge_index[1], jnp.full((pad,), N, jnp.int32)])
  srcp = src.reshape(NW * K, CH)
  dstp = dst.reshape(NW * K, CH)
  z64 = jnp.zeros((NPAD, 64), jnp.float32)
  z32 = jnp.zeros((NPAD, 32), jnp.float32)
  z8 = jnp.zeros((NPAD, 8), jnp.float32)
  ones8 = jnp.ones((CH, 8), jnp.float32)

  grid = N // RBLK

  p1 = pl.pallas_call(
      _tc1_body,
      grid=(grid,),
      in_specs=[_row_spec(128), _full_spec(128, 64), _full_spec(1, 64)],
      out_specs=_row_spec(64),
      out_shape=jax.ShapeDtypeStruct((N, 64), jnp.float32),
  )(x, c1_W1, c1_b1.reshape(1, 64))

  agg1p, cntp = _edge_kernel(64, True)(p1, srcp, dstp, z64, z8, ones8)
  agg1_0 = agg1p[0, :N, :]
  agg1_1 = agg1p[1, :N, :]
  cnt0 = cntp[0, :N, 0:1]
  cnt1 = cntp[1, :N, 0:1]

  h1, p2, inv = pl.pallas_call(
      _tc2_body,
      grid=(grid,),
      in_specs=[_row_spec(64), _row_spec(64), _row_spec(1), _row_spec(1),
                _row_spec(128), _full_spec(64, 64), _full_spec(128, 64),
                _full_spec(1, 64), _full_spec(64, 32), _full_spec(1, 32)],
      out_specs=[_row_spec(64), _row_spec(32), _row_spec(1)],
      out_shape=[jax.ShapeDtypeStruct((N, 64), jnp.float32),
                 jax.ShapeDtypeStruct((N, 32), jnp.float32),
                 jax.ShapeDtypeStruct((N, 1), jnp.float32)],
  )(agg1_0, agg1_1, cnt0, cnt1, x, c1_W2, c1_Wr, c1_b2.reshape(1, 64),
    c2_W1, c2_b1.reshape(1, 32))

  p2p = jnp.concatenate([p2, jnp.zeros((NPAD - N, 32), jnp.float32)])
  (agg2p,) = _edge_kernel(32, False)(p2p, srcp, dstp, z32)
  agg2_0 = agg2p[0, :N, :]
  agg2_1 = agg2p[1, :N, :]

  y = pl.pallas_call(
      _tc3_body,
      grid=(grid,),
      in_specs=[_row_spec(32), _row_spec(32), _row_spec(1), _row_spec(64),
                _full_spec(32, 32), _full_spec(64, 32), _full_spec(1, 32),
                _full_spec(32, 1), _full_spec(1, 1)],
      out_specs=_row_spec(1),
      out_shape=jax.ShapeDtypeStruct((N, 1), jnp.float32),
  )(agg2_0, agg2_1, inv, h1, c2_W2, c2_Wr, c2_b2.reshape(1, 32),
    lin_W, lin_b.reshape(1, 1))

  return y


# speedup vs baseline: 2.6461x; 1.6927x over previous
import functools

import jax
import jax.numpy as jnp
from jax import lax
from jax.experimental import pallas as pl
from jax.experimental.pallas import tpu as pltpu
from jax.experimental.pallas import tpu_sc as plsc

N = 10000
NC = 2
NS = 16
NW = NC * NS
CH = 128
K = 80
K2 = 2 * K
EP = NW * K * CH
RB = 632
NPAD = NS * RB
RBLK = 2000


def _edge_kernel_l1():
  mesh = plsc.VectorSubcoreMesh(
      core_axis_name="c", subcore_axis_name="s", num_cores=NC, num_subcores=NS)
  out_type = [jax.ShapeDtypeStruct((NC, NPAD, 32), jnp.float32),
              jax.ShapeDtypeStruct((NC, NPAD, 1), jnp.float32)]
  scratch = [
      pltpu.VMEM((K2, CH), jnp.int32),
      pltpu.VMEM((K2, CH), jnp.int32),
      pltpu.VMEM((CH, 32), jnp.float32),
      pltpu.VMEM((CH, 32), jnp.float32),
      pltpu.VMEM((CH, 32), jnp.float32),
      pltpu.VMEM((CH, 32), jnp.float32),
      pltpu.VMEM((CH, 1), jnp.float32),
      pltpu.VMEM_SHARED((NPAD, 32), jnp.float32),
      pltpu.VMEM_SHARED((NPAD, 32), jnp.float32),
      pltpu.VMEM_SHARED((NPAD, 1), jnp.float32),
      pltpu.SemaphoreType.DMA,
      pltpu.SemaphoreType.DMA,
      pltpu.SemaphoreType.DMA,
      pltpu.SemaphoreType.DMA,
  ]

  def body(tab_a, tab_b, srcp, dstp, z32, z1, ones_h,
           out_agg, out_cnt,
           src_v, dst_v, rows0, rows1, rows2, rows3, ones_v,
           agg_sh, tab_sh, cnt_sh, sem0, sem1, sem2, sem3):
    c = lax.axis_index("c")
    s = lax.axis_index("s")
    r0 = s * RB
    pltpu.sync_copy(z32.at[pl.ds(r0, RB), :], agg_sh.at[pl.ds(r0, RB), :])
    pltpu.sync_copy(z1.at[pl.ds(r0, RB), :], cnt_sh.at[pl.ds(r0, RB), :])
    pltpu.sync_copy(ones_h, ones_v)

    @pl.when(c == 0)
    def _():
      pltpu.sync_copy(tab_a.at[pl.ds(r0, RB), :], tab_sh.at[pl.ds(r0, RB), :])

    @pl.when(c == 1)
    def _():
      pltpu.sync_copy(tab_b.at[pl.ds(r0, RB), :], tab_sh.at[pl.ds(r0, RB), :])

    plsc.subcore_barrier()

    base = s * K2
    pltpu.sync_copy(srcp.at[pl.ds(base, K2), :], src_v)
    pltpu.sync_copy(dstp.at[pl.ds(base, K2), :], dst_v)

    bufs = (rows0, rows1, rows2, rows3)
    sems = (sem0, sem1, sem2, sem3)
    for b in range(3):
      pltpu.async_copy(tab_sh.at[src_v.at[b]], bufs[b], sems[b])

    def quad(t, carry):
      for b in range(4):
        jj = 4 * t + b
        pltpu.make_async_copy(tab_sh.at[src_v.at[jj]], bufs[b], sems[b]).wait()

        @pl.when(jj + 3 < K2)
        def _():
          bn = (b + 3) % 4
          pltpu.async_copy(tab_sh.at[src_v.at[jj + 3]], bufs[bn], sems[bn])

        pltpu.sync_copy(bufs[b], agg_sh.at[dst_v.at[jj]], add=True)
        do_cnt = lax.select(c == 0, jj < K, jj >= K)

        @pl.when(do_cnt)
        def _():
          pltpu.sync_copy(ones_v, cnt_sh.at[dst_v.at[jj]], add=True)
      return carry

    lax.fori_loop(0, K2 // 4, quad, 0)
    plsc.subcore_barrier()
    pltpu.sync_copy(agg_sh.at[pl.ds(r0, RB), :],
                    out_agg.at[c].at[pl.ds(r0, RB), :])
    pltpu.sync_copy(cnt_sh.at[pl.ds(r0, RB), :],
                    out_cnt.at[c].at[pl.ds(r0, RB), :])

  return pl.kernel(body, out_type=out_type, mesh=mesh, scratch_types=scratch,
                   compiler_params=pltpu.CompilerParams(
                       use_tc_tiling_on_sc=False))


def _edge_kernel_l2(Dm):
  mesh = plsc.VectorSubcoreMesh(
      core_axis_name="c", subcore_axis_name="s", num_cores=NC, num_subcores=NS)
  out_type = jax.ShapeDtypeStruct((NC, NPAD, Dm), jnp.float32)
  scratch = [
      pltpu.VMEM((K, CH), jnp.int32),
      pltpu.VMEM((K, CH), jnp.int32),
      pltpu.VMEM((CH, Dm), jnp.float32),
      pltpu.VMEM((CH, Dm), jnp.float32),
      pltpu.VMEM((CH, Dm), jnp.float32),
      pltpu.VMEM((CH, Dm), jnp.float32),
      pltpu.VMEM_SHARED((NPAD, Dm), jnp.float32),
      pltpu.VMEM_SHARED((NPAD, Dm), jnp.float32),
      pltpu.SemaphoreType.DMA,
      pltpu.SemaphoreType.DMA,
      pltpu.SemaphoreType.DMA,
      pltpu.SemaphoreType.DMA,
  ]

  def body(tab, srcp, dstp, z2, out_agg,
           src_v, dst_v, rows0, rows1, rows2, rows3, agg_sh, tab_sh,
           sem0, sem1, sem2, sem3):
    c = lax.axis_index("c")
    s = lax.axis_index("s")
    wid = c * NS + s
    r0 = s * RB
    pltpu.sync_copy(z2.at[pl.ds(r0, RB), :], agg_sh.at[pl.ds(r0, RB), :])
    pltpu.sync_copy(tab.at[pl.ds(r0, RB), :], tab_sh.at[pl.ds(r0, RB), :])
    plsc.subcore_barrier()

    base = wid * K
    pltpu.sync_copy(srcp.at[pl.ds(base, K), :], src_v)
    pltpu.sync_copy(dstp.at[pl.ds(base, K), :], dst_v)

    bufs = (rows0, rows1, rows2, rows3)
    sems = (sem0, sem1, sem2, sem3)
    for b in range(3):
      pltpu.async_copy(tab_sh.at[src_v.at[b]], bufs[b], sems[b])

    def quad(t, carry):
      for b in range(4):
        jj = 4 * t + b
        pltpu.make_async_copy(tab_sh.at[src_v.at[jj]], bufs[b], sems[b]).wait()

        @pl.when(jj + 3 < K)
        def _():
          bn = (b + 3) % 4
          pltpu.async_copy(tab_sh.at[src_v.at[jj + 3]], bufs[bn], sems[bn])

        pltpu.sync_copy(bufs[b], agg_sh.at[dst_v.at[jj]], add=True)
      return carry

    lax.fori_loop(0, K // 4, quad, 0)
    plsc.subcore_barrier()
    pltpu.sync_copy(agg_sh.at[pl.ds(r0, RB), :],
                    out_agg.at[c].at[pl.ds(r0, RB), :])

  return pl.kernel(body, out_type=out_type, mesh=mesh, scratch_types=scratch,
                   compiler_params=pltpu.CompilerParams(
                       use_tc_tiling_on_sc=False))


def _tc1_body(x, W1, b1, out):
  out[...] = jnp.maximum(
      jnp.dot(x[...], W1[...], preferred_element_type=jnp.float32) + b1[...],
      0.0)


def _tc2_body(agg_a, agg_b, cnt0, cnt1, x, W2a, W2b, Wr, b2, W1n, b1n,
              h1, p2, inv):
  iv = 1.0 / jnp.maximum(cnt0[...] + cnt1[...], 1.0)
  h = jnp.maximum(
      jnp.dot(agg_a[...] * iv, W2a[...], preferred_element_type=jnp.float32)
      + jnp.dot(agg_b[...] * iv, W2b[...], preferred_element_type=jnp.float32)
      + jnp.dot(x[...], Wr[...], preferred_element_type=jnp.float32)
      + b2[...], 0.0)
  h1[...] = h
  p2[...] = jnp.maximum(
      jnp.dot(h, W1n[...], preferred_element_type=jnp.float32) + b1n[...], 0.0)
  inv[...] = iv


def _tc3_body(agg0, agg1, inv, h1, W2, Wr, b2, lW, lb, y):
  mean = (agg0[...] + agg1[...]) * inv[...]
  h = jnp.maximum(
      jnp.dot(mean, W2[...], preferred_element_type=jnp.float32)
      + jnp.dot(h1[...], Wr[...], preferred_element_type=jnp.float32)
      + b2[...], 0.0)
  y[...] = jnp.dot(h, lW[...], preferred_element_type=jnp.float32) + lb[...]


def _row_spec(d):
  return pl.BlockSpec((RBLK, d), lambda i: (i, 0))


def _full_spec(a, b):
  return pl.BlockSpec((a, b), lambda i: (0, 0))


@jax.jit
def kernel(x, edge_index, c1_W1, c1_b1, c1_W2, c1_b2, c1_Wr,
           c2_W1, c2_b1, c2_W2, c2_b2, c2_Wr, lin_W, lin_b):
  E = edge_index.shape[1]
  pad = EP - E
  src = jnp.concatenate([edge_index[0], jnp.zeros((pad,), jnp.int32)])
  dst = jnp.concatenate([edge_index[1], jnp.full((pad,), N, jnp.int32)])
  srcp = src.reshape(NW * K, CH)
  dstp = dst.reshape(NW * K, CH)
  z32 = jnp.zeros((NPAD, 32), jnp.float32)
  z1 = jnp.zeros((NPAD, 1), jnp.float32)
  ones1 = jnp.ones((CH, 1), jnp.float32)

  grid = N // RBLK

  p1 = pl.pallas_call(
      _tc1_body,
      grid=(grid,),
      in_specs=[_row_spec(128), _full_spec(128, 64), _full_spec(1, 64)],
      out_specs=_row_spec(64),
      out_shape=jax.ShapeDtypeStruct((N, 64), jnp.float32),
  )(x, c1_W1, c1_b1.reshape(1, 64))

  p1p = jnp.concatenate([p1, jnp.zeros((NPAD - N, 64), jnp.float32)])
  agg1p, cntp = _edge_kernel_l1()(
      p1p[:, :32], p1p[:, 32:], srcp, dstp, z32, z1, ones1)
  agg1_a = agg1p[0, :N, :]
  agg1_b = agg1p[1, :N, :]
  cnt0 = cntp[0, :N, :]
  cnt1 = cntp[1, :N, :]

  h1, p2, inv = pl.pallas_call(
      _tc2_body,
      grid=(grid,),
      in_specs=[_row_spec(32), _row_spec(32), _row_spec(1), _row_spec(1),
                _row_spec(128), _full_spec(32, 64), _full_spec(32, 64),
                _full_spec(128, 64), _full_spec(1, 64),
                _full_spec(64, 32), _full_spec(1, 32)],
      out_specs=[_row_spec(64), _row_spec(32), _row_spec(1)],
      out_shape=[jax.ShapeDtypeStruct((N, 64), jnp.float32),
                 jax.ShapeDtypeStruct((N, 32), jnp.float32),
                 jax.ShapeDtypeStruct((N, 1), jnp.float32)],
  )(agg1_a, agg1_b, cnt0, cnt1, x, c1_W2[:32, :], c1_W2[32:, :], c1_Wr,
    c1_b2.reshape(1, 64), c2_W1, c2_b1.reshape(1, 32))

  p2p = jnp.concatenate([p2, jnp.zeros((NPAD - N, 32), jnp.float32)])
  agg2p = _edge_kernel_l2(32)(p2p, srcp, dstp, z32)
  agg2_0 = agg2p[0, :N, :]
  agg2_1 = agg2p[1, :N, :]

  y = pl.pallas_call(
      _tc3_body,
      grid=(grid,),
      in_specs=[_row_spec(32), _row_spec(32), _row_spec(1), _row_spec(64),
                _full_spec(32, 32), _full_spec(64, 32), _full_spec(1, 32),
                _full_spec(32, 1), _full_spec(1, 1)],
      out_specs=_row_spec(1),
      out_shape=jax.ShapeDtypeStruct((N, 1), jnp.float32),
  )(agg2_0, agg2_1, inv, h1, c2_W2, c2_Wr, c2_b2.reshape(1, 32),
    lin_W, lin_b.reshape(1, 1))

  return y


# speedup vs baseline: 2.6711x; 1.0094x over previous
import functools

import jax
import jax.numpy as jnp
from jax import lax
from jax.experimental import pallas as pl
from jax.experimental.pallas import tpu as pltpu
from jax.experimental.pallas import tpu_sc as plsc

N = 10000
NC = 2
NS = 16
NW = NC * NS
CH = 128
K = 80
K2 = 2 * K
EP = NW * K * CH
RB = 632
NPAD = NS * RB
RBLK = 2000


def _edge_kernel_l1():
  mesh = plsc.VectorSubcoreMesh(
      core_axis_name="c", subcore_axis_name="s", num_cores=NC, num_subcores=NS)
  out_type = [jax.ShapeDtypeStruct((NC, NPAD, 32), jnp.float32),
              jax.ShapeDtypeStruct((NC, NPAD, 8), jnp.float32)]
  scratch = [
      pltpu.VMEM((K2, CH), jnp.int32),
      pltpu.VMEM((K2, CH), jnp.int32),
      pltpu.VMEM((CH, 32), jnp.float32),
      pltpu.VMEM((CH, 32), jnp.float32),
      pltpu.VMEM((CH, 32), jnp.float32),
      pltpu.VMEM((CH, 32), jnp.float32),
      pltpu.VMEM((CH, 8), jnp.float32),
      pltpu.VMEM_SHARED((NPAD, 32), jnp.float32),
      pltpu.VMEM_SHARED((NPAD, 32), jnp.float32),
      pltpu.VMEM_SHARED((NPAD, 8), jnp.float32),
      pltpu.SemaphoreType.DMA,
      pltpu.SemaphoreType.DMA,
      pltpu.SemaphoreType.DMA,
      pltpu.SemaphoreType.DMA,
  ]

  def body(tab_a, tab_b, srcp, dstp, z32, z1, ones_h,
           out_agg, out_cnt,
           src_v, dst_v, rows0, rows1, rows2, rows3, ones_v,
           agg_sh, tab_sh, cnt_sh, sem0, sem1, sem2, sem3):
    c = lax.axis_index("c")
    s = lax.axis_index("s")
    r0 = s * RB
    pltpu.sync_copy(z32.at[pl.ds(r0, RB), :], agg_sh.at[pl.ds(r0, RB), :])
    pltpu.sync_copy(z1.at[pl.ds(r0, RB), :], cnt_sh.at[pl.ds(r0, RB), :])
    pltpu.sync_copy(ones_h, ones_v)

    @pl.when(c == 0)
    def _():
      pltpu.sync_copy(tab_a.at[pl.ds(r0, RB), :], tab_sh.at[pl.ds(r0, RB), :])

    @pl.when(c == 1)
    def _():
      pltpu.sync_copy(tab_b.at[pl.ds(r0, RB), :], tab_sh.at[pl.ds(r0, RB), :])

    plsc.subcore_barrier()

    base = s * K2
    pltpu.sync_copy(srcp.at[pl.ds(base, K2), :], src_v)
    pltpu.sync_copy(dstp.at[pl.ds(base, K2), :], dst_v)

    bufs = (rows0, rows1, rows2, rows3)
    sems = (sem0, sem1, sem2, sem3)
    for b in range(3):
      pltpu.async_copy(tab_sh.at[src_v.at[b]], bufs[b], sems[b])

    def quad(t, carry):
      for b in range(4):
        jj = 4 * t + b
        pltpu.make_async_copy(tab_sh.at[src_v.at[jj]], bufs[b], sems[b]).wait()

        @pl.when(jj + 3 < K2)
        def _():
          bn = (b + 3) % 4
          pltpu.async_copy(tab_sh.at[src_v.at[jj + 3]], bufs[bn], sems[bn])

        pltpu.sync_copy(bufs[b], agg_sh.at[dst_v.at[jj]], add=True)
        do_cnt = lax.select(c == 0, jj < K, jj >= K)

        @pl.when(do_cnt)
        def _():
          pltpu.sync_copy(ones_v, cnt_sh.at[dst_v.at[jj]], add=True)
      return carry

    lax.fori_loop(0, K2 // 4, quad, 0)
    plsc.subcore_barrier()
    pltpu.sync_copy(agg_sh.at[pl.ds(r0, RB), :],
                    out_agg.at[c].at[pl.ds(r0, RB), :])
    pltpu.sync_copy(cnt_sh.at[pl.ds(r0, RB), :],
                    out_cnt.at[c].at[pl.ds(r0, RB), :])

  return pl.kernel(body, out_type=out_type, mesh=mesh, scratch_types=scratch,
                   compiler_params=pltpu.CompilerParams(
                       use_tc_tiling_on_sc=False))


def _edge_kernel_l2(Dm):
  mesh = plsc.VectorSubcoreMesh(
      core_axis_name="c", subcore_axis_name="s", num_cores=NC, num_subcores=NS)
  out_type = jax.ShapeDtypeStruct((NC, NPAD, Dm), jnp.float32)
  scratch = [
      pltpu.VMEM((K, CH), jnp.int32),
      pltpu.VMEM((K, CH), jnp.int32),
      pltpu.VMEM((CH, Dm), jnp.float32),
      pltpu.VMEM((CH, Dm), jnp.float32),
      pltpu.VMEM((CH, Dm), jnp.float32),
      pltpu.VMEM((CH, Dm), jnp.float32),
      pltpu.VMEM_SHARED((NPAD, Dm), jnp.float32),
      pltpu.VMEM_SHARED((NPAD, Dm), jnp.float32),
      pltpu.SemaphoreType.DMA,
      pltpu.SemaphoreType.DMA,
      pltpu.SemaphoreType.DMA,
      pltpu.SemaphoreType.DMA,
  ]

  def body(tab, srcp, dstp, z2, out_agg,
           src_v, dst_v, rows0, rows1, rows2, rows3, agg_sh, tab_sh,
           sem0, sem1, sem2, sem3):
    c = lax.axis_index("c")
    s = lax.axis_index("s")
    wid = c * NS + s
    r0 = s * RB
    pltpu.sync_copy(z2.at[pl.ds(r0, RB), :], agg_sh.at[pl.ds(r0, RB), :])
    pltpu.sync_copy(tab.at[pl.ds(r0, RB), :], tab_sh.at[pl.ds(r0, RB), :])
    plsc.subcore_barrier()

    base = wid * K
    pltpu.sync_copy(srcp.at[pl.ds(base, K), :], src_v)
    pltpu.sync_copy(dstp.at[pl.ds(base, K), :], dst_v)

    bufs = (rows0, rows1, rows2, rows3)
    sems = (sem0, sem1, sem2, sem3)
    for b in range(3):
      pltpu.async_copy(tab_sh.at[src_v.at[b]], bufs[b], sems[b])

    def quad(t, carry):
      for b in range(4):
        jj = 4 * t + b
        pltpu.make_async_copy(tab_sh.at[src_v.at[jj]], bufs[b], sems[b]).wait()

        @pl.when(jj + 3 < K)
        def _():
          bn = (b + 3) % 4
          pltpu.async_copy(tab_sh.at[src_v.at[jj + 3]], bufs[bn], sems[bn])

        pltpu.sync_copy(bufs[b], agg_sh.at[dst_v.at[jj]], add=True)
      return carry

    lax.fori_loop(0, K // 4, quad, 0)
    plsc.subcore_barrier()
    pltpu.sync_copy(agg_sh.at[pl.ds(r0, RB), :],
                    out_agg.at[c].at[pl.ds(r0, RB), :])

  return pl.kernel(body, out_type=out_type, mesh=mesh, scratch_types=scratch,
                   compiler_params=pltpu.CompilerParams(
                       use_tc_tiling_on_sc=False))


def _tc1_body(x, W1, b1, out):
  out[...] = jnp.maximum(
      jnp.dot(x[...], W1[...], preferred_element_type=jnp.float32) + b1[...],
      0.0)


def _tc2_body(agg_a, agg_b, cnt0, cnt1, x, W2a, W2b, Wr, b2, W1n, b1n,
              h1, p2, inv):
  iv = 1.0 / jnp.maximum(cnt0[...] + cnt1[...], 1.0)
  h = jnp.maximum(
      jnp.dot(agg_a[...] * iv, W2a[...], preferred_element_type=jnp.float32)
      + jnp.dot(agg_b[...] * iv, W2b[...], preferred_element_type=jnp.float32)
      + jnp.dot(x[...], Wr[...], preferred_element_type=jnp.float32)
      + b2[...], 0.0)
  h1[...] = h
  p2[...] = jnp.maximum(
      jnp.dot(h, W1n[...], preferred_element_type=jnp.float32) + b1n[...], 0.0)
  inv[...] = iv


def _tc3_body(agg0, agg1, inv, h1, W2, Wr, b2, lW, lb, y):
  mean = (agg0[...] + agg1[...]) * inv[...]
  h = jnp.maximum(
      jnp.dot(mean, W2[...], preferred_element_type=jnp.float32)
      + jnp.dot(h1[...], Wr[...], preferred_element_type=jnp.float32)
      + b2[...], 0.0)
  y[...] = jnp.dot(h, lW[...], preferred_element_type=jnp.float32) + lb[...]


def _row_spec(d):
  return pl.BlockSpec((RBLK, d), lambda i: (i, 0))


def _full_spec(a, b):
  return pl.BlockSpec((a, b), lambda i: (0, 0))


@jax.jit
def kernel(x, edge_index, c1_W1, c1_b1, c1_W2, c1_b2, c1_Wr,
           c2_W1, c2_b1, c2_W2, c2_b2, c2_Wr, lin_W, lin_b):
  E = edge_index.shape[1]
  pad = EP - E
  src = jnp.concatenate([edge_index[0], jnp.zeros((pad,), jnp.int32)])
  dst = jnp.concatenate([edge_index[1], jnp.full((pad,), N, jnp.int32)])
  srcp = src.reshape(NW * K, CH)
  dstp = dst.reshape(NW * K, CH)
  z32 = jnp.zeros((NPAD, 32), jnp.float32)
  z1 = jnp.zeros((NPAD, 8), jnp.float32)
  ones1 = jnp.ones((CH, 8), jnp.float32)

  grid = N // RBLK

  p1 = pl.pallas_call(
      _tc1_body,
      grid=(grid,),
      in_specs=[_row_spec(128), _full_spec(128, 64), _full_spec(1, 64)],
      out_specs=_row_spec(64),
      out_shape=jax.ShapeDtypeStruct((N, 64), jnp.float32),
  )(x, c1_W1, c1_b1.reshape(1, 64))

  p1p = jnp.concatenate([p1, jnp.zeros((NPAD - N, 64), jnp.float32)])
  agg1p, cntp = _edge_kernel_l1()(
      p1p[:, :32], p1p[:, 32:], srcp, dstp, z32, z1, ones1)
  agg1_a = agg1p[0, :N, :]
  agg1_b = agg1p[1, :N, :]
  cnt0 = cntp[0, :N, 0:1]
  cnt1 = cntp[1, :N, 0:1]

  h1, p2, inv = pl.pallas_call(
      _tc2_body,
      grid=(grid,),
      in_specs=[_row_spec(32), _row_spec(32), _row_spec(1), _row_spec(1),
                _row_spec(128), _full_spec(32, 64), _full_spec(32, 64),
                _full_spec(128, 64), _full_spec(1, 64),
                _full_spec(64, 32), _full_spec(1, 32)],
      out_specs=[_row_spec(64), _row_spec(32), _row_spec(1)],
      out_shape=[jax.ShapeDtypeStruct((N, 64), jnp.float32),
                 jax.ShapeDtypeStruct((N, 32), jnp.float32),
                 jax.ShapeDtypeStruct((N, 1), jnp.float32)],
  )(agg1_a, agg1_b, cnt0, cnt1, x, c1_W2[:32, :], c1_W2[32:, :], c1_Wr,
    c1_b2.reshape(1, 64), c2_W1, c2_b1.reshape(1, 32))

  p2p = jnp.concatenate([p2, jnp.zeros((NPAD - N, 32), jnp.float32)])
  agg2p = _edge_kernel_l2(32)(p2p, srcp, dstp, z32)
  agg2_0 = agg2p[0, :N, :]
  agg2_1 = agg2p[1, :N, :]

  y = pl.pallas_call(
      _tc3_body,
      grid=(grid,),
      in_specs=[_row_spec(32), _row_spec(32), _row_spec(1), _row_spec(64),
                _full_spec(32, 32), _full_spec(64, 32), _full_spec(1, 32),
                _full_spec(32, 1), _full_spec(1, 1)],
      out_specs=_row_spec(1),
      out_shape=jax.ShapeDtypeStruct((N, 1), jnp.float32),
  )(agg2_0, agg2_1, inv, h1, c2_W2, c2_Wr, c2_b2.reshape(1, 32),
    lin_W, lin_b.reshape(1, 1))

  return y
